# baseline (device time: 14816 ns/iter reference)
import jax
import jax.numpy as jnp
from jax import lax
from jax.experimental import pallas as pl
from jax.experimental.pallas import tpu as pltpu

C = 4


def kernel(dy, W):
    m, k = dy.shape
    d, _ = W.shape
    mb = m // 2
    rc = mb // C

    dy = pltpu.with_memory_space_constraint(dy, pltpu.MemorySpace.HBM)
    W = pltpu.with_memory_space_constraint(W, pltpu.MemorySpace.HBM)

    def body(
        dy_ref,
        w_ref,
        out_ref,
        wv,
        dyv,
        pbuf,
        xbuf,
        xrecv,
        ybuf,
        yrecv,
        rbuf,
        prbuf,
        w_sem,
        dy_sem,
        out_sems,
        pout_sems,
        x_send_sems,
        x_recv_sems,
        y_send_sems,
        y_recv_sems,
    ):
        my_x = lax.axis_index("x")
        my_y = lax.axis_index("y")
        base = my_y * mb
        peer_base = (1 - my_y) * mb

        w_copy = pltpu.make_async_copy(w_ref, wv, w_sem)
        w_copy.start()
        dy_copy = pltpu.make_async_copy(
            dy_ref.at[pl.ds(base, mb)], dyv, dy_sem
        )
        dy_copy.start()

        w_copy.wait()
        dy_copy.wait()
        pbuf[:, :] = lax.dot_general(
            dyv[:, :],
            wv[:, :],
            dimension_numbers=(((1,), (1,)), ((), ())),
            preferred_element_type=jnp.float32,
        )
        for c in range(C):
            xbuf[c, :, :] = pbuf[pl.ds(c * rc, rc), :].astype(jnp.bfloat16)

        barrier_sem = pltpu.get_barrier_semaphore()
        pl.semaphore_signal(
            barrier_sem, inc=1, device_id=(1 - my_x, my_y),
            device_id_type=pl.DeviceIdType.MESH,
        )
        pl.semaphore_signal(
            barrier_sem, inc=1, device_id=(my_x, 1 - my_y),
            device_id_type=pl.DeviceIdType.MESH,
        )
        pl.semaphore_wait(barrier_sem, 2)

        x_rdmas = []
        for c in range(C):
            r = pltpu.make_async_remote_copy(
                src_ref=xbuf.at[c],
                dst_ref=xrecv.at[c],
                send_sem=x_send_sems.at[c],
                recv_sem=x_recv_sems.at[c],
                device_id=(1 - my_x, my_y),
                device_id_type=pl.DeviceIdType.MESH,
            )
            r.start()
            x_rdmas.append(r)

        y_rdmas = []
        out_copies = []
        for c in range(C):
            x_rdmas[c].wait_recv()
            red = pbuf[pl.ds(c * rc, rc), :] + xrecv[c].astype(jnp.float32)
            rbuf[c, :, :] = red
            ybuf[c, :, :] = red.astype(jnp.bfloat16)
            oc = pltpu.make_async_copy(
                rbuf.at[c], out_ref.at[pl.ds(base + c * rc, rc)],
                out_sems.at[c],
            )
            oc.start()
            out_copies.append(oc)
            s = pltpu.make_async_remote_copy(
                src_ref=ybuf.at[c],
                dst_ref=yrecv.at[c],
                send_sem=y_send_sems.at[c],
                recv_sem=y_recv_sems.at[c],
                device_id=(my_x, 1 - my_y),
                device_id_type=pl.DeviceIdType.MESH,
            )
            s.start()
            y_rdmas.append(s)

        pout_copies = []
        for c in range(C):
            y_rdmas[c].wait_recv()
            prbuf[c, :, :] = yrecv[c].astype(jnp.float32)
            pc = pltpu.make_async_copy(
                prbuf.at[c], out_ref.at[pl.ds(peer_base + c * rc, rc)],
                pout_sems.at[c],
            )
            pc.start()
            pout_copies.append(pc)

        for c in range(C):
            out_copies[c].wait()
            pout_copies[c].wait()
            x_rdmas[c].wait_send()
            y_rdmas[c].wait_send()

    return pl.pallas_call(
        body,
        out_shape=jax.ShapeDtypeStruct((m, d), jnp.float32),
        in_specs=[
            pl.BlockSpec(memory_space=pltpu.MemorySpace.HBM),
            pl.BlockSpec(memory_space=pltpu.MemorySpace.HBM),
        ],
        out_specs=pl.BlockSpec(memory_space=pltpu.MemorySpace.HBM),
        scratch_shapes=[
            pltpu.VMEM((d, k), jnp.float32),
            pltpu.VMEM((mb, k), jnp.float32),
            pltpu.VMEM((mb, d), jnp.float32),
            pltpu.VMEM((C, rc, d), jnp.bfloat16),
            pltpu.VMEM((C, rc, d), jnp.bfloat16),
            pltpu.VMEM((C, rc, d), jnp.bfloat16),
            pltpu.VMEM((C, rc, d), jnp.bfloat16),
            pltpu.VMEM((C, rc, d), jnp.float32),
            pltpu.VMEM((C, rc, d), jnp.float32),
            pltpu.SemaphoreType.DMA,
            pltpu.SemaphoreType.DMA,
            pltpu.SemaphoreType.DMA((C,)),
            pltpu.SemaphoreType.DMA((C,)),
            pltpu.SemaphoreType.DMA((C,)),
            pltpu.SemaphoreType.DMA((C,)),
            pltpu.SemaphoreType.DMA((C,)),
            pltpu.SemaphoreType.DMA((C,)),
        ],
        compiler_params=pltpu.CompilerParams(collective_id=0),
    )(dy, W)


# device time: 14409 ns/iter; 1.0282x vs baseline; 1.0282x over previous
import jax
import jax.numpy as jnp
from jax import lax
from jax.experimental import pallas as pl
from jax.experimental.pallas import tpu as pltpu

C = 4


def kernel(dy, W):
    m, k = dy.shape
    d, _ = W.shape
    mb = m // 2
    rc = mb // C

    dy = pltpu.with_memory_space_constraint(dy, pltpu.MemorySpace.HBM)
    W = pltpu.with_memory_space_constraint(W, pltpu.MemorySpace.HBM)

    def body(
        dy_ref,
        w_ref,
        out_ref,
        wv,
        dyv,
        pf32,
        pbuf,
        xrecv,
        ybuf,
        yrecv,
        w_sem,
        dy_sem,
        x_send_sems,
        x_recv_sems,
        y_send_sems,
        y_recv_sems,
    ):
        my_x = lax.axis_index("x")
        my_y = lax.axis_index("y")
        base = my_y * mb
        peer_base = (1 - my_y) * mb

        w_copy = pltpu.make_async_copy(w_ref, wv, w_sem)
        w_copy.start()
        dy_copy = pltpu.make_async_copy(
            dy_ref.at[pl.ds(base, mb)], dyv, dy_sem
        )
        dy_copy.start()

        w_copy.wait()
        dy_copy.wait()
        pf32[:, :] = lax.dot_general(
            dyv[:, :],
            wv[:, :],
            dimension_numbers=(((1,), (1,)), ((), ())),
            preferred_element_type=jnp.float32,
        )
        pbuf[:, :] = pf32[:, :].astype(jnp.bfloat16)

        barrier_sem = pltpu.get_barrier_semaphore()
        pl.semaphore_signal(
            barrier_sem, inc=1, device_id=(1 - my_x, my_y),
            device_id_type=pl.DeviceIdType.MESH,
        )
        pl.semaphore_signal(
            barrier_sem, inc=1, device_id=(my_x, 1 - my_y),
            device_id_type=pl.DeviceIdType.MESH,
        )
        pl.semaphore_wait(barrier_sem, 2)

        x_rdmas = []
        for c in range(C):
            r = pltpu.make_async_remote_copy(
                src_ref=pbuf.at[pl.ds(c * rc, rc)],
                dst_ref=xrecv.at[c],
                send_sem=x_send_sems.at[c],
                recv_sem=x_recv_sems.at[c],
                device_id=(1 - my_x, my_y),
                device_id_type=pl.DeviceIdType.MESH,
            )
            r.start()
            x_rdmas.append(r)

        y_rdmas = []
        for c in range(C):
            x_rdmas[c].wait_recv()
            sum_bf = pbuf[pl.ds(c * rc, rc), :] + xrecv[c]
            ybuf[c, :, :] = sum_bf
            out_ref[pl.ds(base + c * rc, rc), :] = sum_bf.astype(jnp.float32)
            s = pltpu.make_async_remote_copy(
                src_ref=ybuf.at[c],
                dst_ref=yrecv.at[c],
                send_sem=y_send_sems.at[c],
                recv_sem=y_recv_sems.at[c],
                device_id=(my_x, 1 - my_y),
                device_id_type=pl.DeviceIdType.MESH,
            )
            s.start()
            y_rdmas.append(s)

        for c in range(C):
            y_rdmas[c].wait_recv()
            out_ref[pl.ds(peer_base + c * rc, rc), :] = yrecv[c].astype(
                jnp.float32
            )

        for c in range(C):
            x_rdmas[c].wait_send()
            y_rdmas[c].wait_send()

    return pl.pallas_call(
        body,
        out_shape=jax.ShapeDtypeStruct((m, d), jnp.float32),
        in_specs=[
            pl.BlockSpec(memory_space=pltpu.MemorySpace.HBM),
            pl.BlockSpec(memory_space=pltpu.MemorySpace.HBM),
        ],
        out_specs=pl.BlockSpec(memory_space=pltpu.VMEM),
        scratch_shapes=[
            pltpu.VMEM((d, k), jnp.float32),
            pltpu.VMEM((mb, k), jnp.float32),
            pltpu.VMEM((mb, d), jnp.float32),
            pltpu.VMEM((mb, d), jnp.bfloat16),
            pltpu.VMEM((C, rc, d), jnp.bfloat16),
            pltpu.VMEM((C, rc, d), jnp.bfloat16),
            pltpu.VMEM((C, rc, d), jnp.bfloat16),
            pltpu.SemaphoreType.DMA,
            pltpu.SemaphoreType.DMA,
            pltpu.SemaphoreType.DMA((C,)),
            pltpu.SemaphoreType.DMA((C,)),
            pltpu.SemaphoreType.DMA((C,)),
            pltpu.SemaphoreType.DMA((C,)),
        ],
        compiler_params=pltpu.CompilerParams(collective_id=0),
    )(dy, W)
